# baseline (device time: 21844 ns/iter reference)
import jax
import jax.numpy as jnp
from jax import lax
from jax.experimental import pallas as pl
from jax.experimental.pallas import tpu as pltpu


def kernel(dy, W):
    m, k = dy.shape
    d = W.shape[0]

    def body(dy_ref, w_ref, out_ref, acc_ref, recv_ref, send_sem, recv_sem):
        my_x = lax.axis_index("x")
        my_y = lax.axis_index("y")
        my_z = lax.axis_index("z")
        peer = (1 - my_x, my_y, my_z)

        barrier_sem = pltpu.get_barrier_semaphore()
        pl.semaphore_signal(
            barrier_sem, inc=1,
            device_id=peer, device_id_type=pl.DeviceIdType.MESH,
        )
        pl.semaphore_wait(barrier_sem, 1)

        acc_ref[:, :] = lax.dot_general(
            dy_ref[:, :], w_ref[:, :],
            (((1,), (1,)), ((), ())),
            preferred_element_type=jnp.float32,
        )

        rdma = pltpu.make_async_remote_copy(
            src_ref=acc_ref,
            dst_ref=recv_ref,
            send_sem=send_sem,
            recv_sem=recv_sem,
            device_id=peer,
            device_id_type=pl.DeviceIdType.MESH,
        )
        rdma.start()
        rdma.wait()

        out_ref[:, :] = acc_ref[:, :] + recv_ref[:, :]

    return pl.pallas_call(
        body,
        out_shape=jax.ShapeDtypeStruct((m, d), jnp.float32),
        in_specs=[
            pl.BlockSpec(memory_space=pltpu.VMEM),
            pl.BlockSpec(memory_space=pltpu.VMEM),
        ],
        out_specs=pl.BlockSpec(memory_space=pltpu.VMEM),
        scratch_shapes=[
            pltpu.VMEM((m, d), jnp.float32),
            pltpu.VMEM((m, d), jnp.float32),
            pltpu.SemaphoreType.DMA,
            pltpu.SemaphoreType.DMA,
        ],
        compiler_params=pltpu.CompilerParams(collective_id=0),
    )(dy, W)


# device time: 15712 ns/iter; 1.3903x vs baseline; 1.3903x over previous
import jax
import jax.numpy as jnp
from jax import lax
from jax.experimental import pallas as pl
from jax.experimental.pallas import tpu as pltpu

NCHUNK = 4


def kernel(dy, W):
    m, k = dy.shape
    d = W.shape[0]
    mc = m // NCHUNK

    def body(dy_ref, w_ref, out_ref, w_bf_ref, send_ref, recv_ref,
             send_sems, recv_sems):
        my_x = lax.axis_index("x")
        my_y = lax.axis_index("y")
        my_z = lax.axis_index("z")
        peer = (1 - my_x, my_y, my_z)

        barrier_sem = pltpu.get_barrier_semaphore()
        pl.semaphore_signal(
            barrier_sem, inc=1,
            device_id=peer, device_id_type=pl.DeviceIdType.MESH,
        )
        pl.semaphore_wait(barrier_sem, 1)

        w_bf_ref[:, :] = w_ref[:, :].astype(jnp.bfloat16)

        rdmas = []
        for c in range(NCHUNK):
            rows = pl.ds(c * mc, mc)
            acc = lax.dot_general(
                dy_ref[rows, :].astype(jnp.bfloat16), w_bf_ref[:, :],
                (((1,), (1,)), ((), ())),
                preferred_element_type=jnp.float32,
            )
            out_ref[rows, :] = acc
            send_ref[rows, :] = acc.astype(jnp.bfloat16)
            rdma = pltpu.make_async_remote_copy(
                src_ref=send_ref.at[rows, :],
                dst_ref=recv_ref.at[rows, :],
                send_sem=send_sems.at[c],
                recv_sem=recv_sems.at[c],
                device_id=peer,
                device_id_type=pl.DeviceIdType.MESH,
            )
            rdma.start()
            rdmas.append(rdma)

        for rdma in rdmas:
            rdma.wait()

        out_ref[:, :] = out_ref[:, :] + recv_ref[:, :].astype(jnp.float32)

    return pl.pallas_call(
        body,
        out_shape=jax.ShapeDtypeStruct((m, d), jnp.float32),
        in_specs=[
            pl.BlockSpec(memory_space=pltpu.VMEM),
            pl.BlockSpec(memory_space=pltpu.VMEM),
        ],
        out_specs=pl.BlockSpec(memory_space=pltpu.VMEM),
        scratch_shapes=[
            pltpu.VMEM((d, k), jnp.bfloat16),
            pltpu.VMEM((m, d), jnp.bfloat16),
            pltpu.VMEM((m, d), jnp.bfloat16),
            pltpu.SemaphoreType.DMA((NCHUNK,)),
            pltpu.SemaphoreType.DMA((NCHUNK,)),
        ],
        compiler_params=pltpu.CompilerParams(collective_id=0),
    )(dy, W)


# device time: 15591 ns/iter; 1.4011x vs baseline; 1.0078x over previous
import jax
import jax.numpy as jnp
from jax import lax
from jax.experimental import pallas as pl
from jax.experimental.pallas import tpu as pltpu

NCHUNK = 4


def kernel(dy, W):
    m, k = dy.shape
    d = W.shape[0]
    mc = m // NCHUNK

    def body(dy_ref, w_ref, out_ref, send_ref, recv_ref, send_sems, recv_sems):
        my_x = lax.axis_index("x")
        my_y = lax.axis_index("y")
        my_z = lax.axis_index("z")
        peer = (1 - my_x, my_y, my_z)

        barrier_sem = pltpu.get_barrier_semaphore()
        pl.semaphore_signal(
            barrier_sem, inc=1,
            device_id=peer, device_id_type=pl.DeviceIdType.MESH,
        )
        pl.semaphore_wait(barrier_sem, 1)

        rdmas = []
        for c in range(NCHUNK):
            rows = pl.ds(c * mc, mc)
            acc = lax.dot_general(
                dy_ref[rows, :], w_ref[:, :],
                (((1,), (1,)), ((), ())),
                preferred_element_type=jnp.float32,
            )
            out_ref[rows, :] = acc
            send_ref[rows, :] = acc.astype(jnp.bfloat16)
            rdma = pltpu.make_async_remote_copy(
                src_ref=send_ref.at[rows, :],
                dst_ref=recv_ref.at[rows, :],
                send_sem=send_sems.at[c],
                recv_sem=recv_sems.at[c],
                device_id=peer,
                device_id_type=pl.DeviceIdType.MESH,
            )
            rdma.start()
            rdmas.append(rdma)

        for c, rdma in enumerate(rdmas):
            rows = pl.ds(c * mc, mc)
            rdma.wait_recv()
            out_ref[rows, :] = out_ref[rows, :] + recv_ref[rows, :].astype(
                jnp.float32
            )
        for rdma in rdmas:
            rdma.wait_send()

    return pl.pallas_call(
        body,
        out_shape=jax.ShapeDtypeStruct((m, d), jnp.float32),
        in_specs=[
            pl.BlockSpec(memory_space=pltpu.VMEM),
            pl.BlockSpec(memory_space=pltpu.VMEM),
        ],
        out_specs=pl.BlockSpec(memory_space=pltpu.VMEM),
        scratch_shapes=[
            pltpu.VMEM((m, d), jnp.bfloat16),
            pltpu.VMEM((m, d), jnp.bfloat16),
            pltpu.SemaphoreType.DMA((NCHUNK,)),
            pltpu.SemaphoreType.DMA((NCHUNK,)),
        ],
        compiler_params=pltpu.CompilerParams(collective_id=0),
    )(dy, W)
